# baseline (speedup 1.0000x reference)
import functools

import jax
import jax.numpy as jnp
from jax import lax
from jax.experimental import pallas as pl
from jax.experimental.pallas import tpu as pltpu
from jax.experimental.pallas import tpu_sc as plsc

B = 4096
L = 200
D = 32
V = 1000000
NC = 2
NS = 16
NW = NC * NS
SEQ_PER_W = B // NW
C0, C1 = 104, 96
UNROLL = 8

VCH = 512
RP = 40
NTILECH = V // VCH
VTAIL = V - NTILECH * VCH


def _fmt_body(tblT_hbm, tail_hbm, out_hbm, in0, in1, outb0, outb1,
              isem0, isem1, osem0, osem1):
    wid = lax.axis_index("s") * NC + lax.axis_index("c")

    lane = lax.iota(jnp.int32, 16)
    lane32 = lane * D

    in_bufs = (in0, in1)
    out_bufs = (outb0, outb1)
    isems = (isem0, isem1)
    osems = (osem0, osem1)

    def issue_read(c, inb, isem):
        v0 = c * VCH
        for dblk in range(4):
            for vblk in range(VCH // 128):
                pltpu.async_copy(
                    tblT_hbm.at[pl.ds(dblk * 8, 8),
                                pl.ds(v0 + vblk * 128, 128)],
                    inb.at[pl.ds(dblk * 8, 8), pl.ds(vblk * 128, 128)],
                    isem)

    def wait_read(inb, isem):
        for _ in range(4 * (VCH // 128)):
            pltpu.make_async_copy(
                tblT_hbm.at[pl.ds(0, 8), pl.ds(0, 128)],
                inb.at[pl.ds(0, 8), pl.ds(0, 128)], isem).wait()

    lane_rp = lane * RP

    def transpose_chunk(inb, outb):
        def tbody(g, carry):
            base = g * 16
            idx = lane_rp + base * RP
            for d in range(D):
                vals = inb[d, pl.ds(base, 16)]
                plsc.store_scatter(outb, [idx + d], vals)
            return carry
        lax.fori_loop(0, VCH // 16, tbody, 0)

    issue_read(wid, in0, isem0)
    issue_read(NW + wid, in1, isem1)

    def round_body(k, carry):
        for p in range(2):
            c = (2 * k + p) * NW + wid
            inb, outb = in_bufs[p], out_bufs[p]
            isem, osem = isems[p], osems[p]

            @pl.when(c < NTILECH)
            def _():
                wait_read(inb, isem)

                @pl.when(k > 0)
                def _():
                    pltpu.make_async_copy(
                        outb, out_hbm.at[pl.ds(0, VCH * RP)], osem).wait()

                transpose_chunk(inb, outb)
                pltpu.async_copy(outb, out_hbm.at[pl.ds(c * VCH * RP,
                                                        VCH * RP)], osem)

                @pl.when(c + 2 * NW < NTILECH)
                def _():
                    issue_read(c + 2 * NW, inb, isem)
        return carry

    nrounds = (NTILECH + 2 * NW - 1) // (2 * NW)
    lax.fori_loop(0, nrounds, round_body, 0)

    for p in range(2):
        pltpu.make_async_copy(out_bufs[p], out_hbm.at[pl.ds(0, VCH * RP)],
                              osems[p]).wait()

    @pl.when(wid == 0)
    def _():
        pltpu.sync_copy(tail_hbm, outb1.at[pl.ds(0, VTAIL * D)])

        def tailb(t, carry):
            for j in range(8):
                v = t * 8 + j
                outb0[pl.ds(v * RP, 16)] = outb1[pl.ds(v * D, 16)]
                outb0[pl.ds(v * RP + 16, 16)] = outb1[pl.ds(v * D + 16,
                                                            16)]
            return carry
        lax.fori_loop(0, VTAIL // 8, tailb, 0)
        pltpu.sync_copy(outb0.at[pl.ds(0, VTAIL * RP)],
                        out_hbm.at[pl.ds(NTILECH * VCH * RP, VTAIL * RP)])


@functools.partial(
    pl.kernel,
    out_type=jax.ShapeDtypeStruct((V * RP,), jnp.float32),
    mesh=plsc.VectorSubcoreMesh(core_axis_name="c", subcore_axis_name="s",
                                num_cores=NC, num_subcores=NS),
    compiler_params=pltpu.CompilerParams(needs_layout_passes=False,
                                         use_tc_tiling_on_sc=True),
    scratch_types=[
        pltpu.VMEM((D, VCH), jnp.float32),
        pltpu.VMEM((D, VCH), jnp.float32),
        pltpu.VMEM((VCH * RP,), jnp.float32),
        pltpu.VMEM((VCH * RP,), jnp.float32),
        pltpu.SemaphoreType.DMA,
        pltpu.SemaphoreType.DMA,
        pltpu.SemaphoreType.DMA,
        pltpu.SemaphoreType.DMA,
    ],
)
def _table_to_rowmajor(tblT_hbm, tail_hbm, out_hbm, *rest):
    _fmt_body(tblT_hbm, tail_hbm, out_hbm, *rest)


LCH = 8
NLCH = L // LCH


def _sc_body(x_hbm, table_hbm, emb_hbm, len_hbm,
             idx_v, gidx, rows0, rows1, rows2, rows3, otile, len_v,
             gsem0, gsem1, gsem2, gsem3, osem):
    wid = lax.axis_index("s") * NC + lax.axis_index("c")
    b0 = wid * SEQ_PER_W

    pltpu.sync_copy(x_hbm.at[pl.ds(b0, SEQ_PER_W)], idx_v)

    lane = lax.iota(jnp.int32, 16)
    lane0 = lane == 0
    tail_mask = lane >= 8

    rows_bufs = (rows0, rows1, rows2, rows3)
    gsems = (gsem0, gsem1, gsem2, gsem3)

    def stage_chunk(c):
        def sbody(t, carry):
            lp = t // 8
            g = t - lp * 8
            v = plsc.load_gather(
                idx_v, [g * 16 + lane,
                        jnp.full((16,), c * LCH + lp, jnp.int32)])
            gidx[pl.ds(t * 16, 16)] = v
            return carry
        lax.fori_loop(0, 64, sbody, 0)

    def issue_gather(e, rows, gsem):
        pltpu.async_copy(table_hbm.at[gidx.at[pl.ds(e * 128, 128)]],
                         rows, gsem)

    def transpose_eighth(e, rows):
        esplat = jnp.full((16,), e, jnp.int32)

        def gbody(g, carry):
            for j in range(8):
                t = g * 8 + j
                v0 = rows[t, pl.ds(0, 16)]
                v1 = rows[t, pl.ds(16, 16)]
                tsplat = jnp.full((16,), t, jnp.int32)
                plsc.store_scatter(otile, [lane, esplat, tsplat], v0)
                plsc.store_scatter(otile, [lane + 16, esplat, tsplat],
                                   v1)
            return carry
        lax.fori_loop(0, 16, gbody, 0)

    def count_lengths():
        def cbody(b, carry):
            cnt = jnp.zeros((16,), jnp.int32)
            for j in range(12):
                v = idx_v[b, pl.ds(j * 16, 16)]
                cnt = cnt + (v != 0).astype(jnp.int32)
            v = idx_v[b, pl.ds(184, 16)]
            cnt = cnt + ((v != 0) & tail_mask).astype(jnp.int32)
            total = jnp.sum(cnt)
            plsc.store_scatter(len_v, [jnp.full((16,), b, jnp.int32)],
                               jnp.full((16,), total, jnp.int32),
                               mask=lane0)
            return carry
        lax.fori_loop(0, SEQ_PER_W, cbody, 0)

    def chunk_body(c, carry):
        stage_chunk(c)

        @pl.when(c > 0)
        def _():
            pltpu.make_async_copy(
                otile.at[:, :, pl.ds(0, 128)],
                emb_hbm.at[:, pl.ds(0, LCH), pl.ds(b0, 128)],
                osem).wait()

        for k in range(4):
            issue_gather(k, rows_bufs[k], gsems[k])

        def pair_body(i, carry2):
            for k in range(4):
                e = i * 4 + k
                rows, gsem = rows_bufs[k], gsems[k]
                pltpu.make_async_copy(table_hbm.at[pl.ds(0, 128)], rows,
                                      gsem).wait()
                transpose_eighth(e, rows)

                @pl.when(e + 4 < LCH)
                def _():
                    issue_gather(e + 4, rows, gsem)
            return carry2
        lax.fori_loop(0, LCH // 4, pair_body, 0)

        for d in range(D):
            pltpu.async_copy(
                otile.at[d, pl.ds(0, LCH), pl.ds(0, 128)],
                emb_hbm.at[d, pl.ds(c * LCH, LCH), pl.ds(b0, 128)],
                osem)
        return carry

    lax.fori_loop(0, NLCH, chunk_body, 0)

    pltpu.make_async_copy(
        otile.at[:, :, pl.ds(0, 128)],
        emb_hbm.at[:, pl.ds(0, LCH), pl.ds(b0, 128)], osem).wait()
    count_lengths()
    pltpu.sync_copy(len_v, len_hbm.at[pl.ds(b0, SEQ_PER_W)])


@functools.partial(
    pl.kernel,
    out_type=(jax.ShapeDtypeStruct((D, L, B), jnp.float32),
              jax.ShapeDtypeStruct((B,), jnp.int32)),
    mesh=plsc.VectorSubcoreMesh(core_axis_name="c", subcore_axis_name="s",
                                num_cores=NC, num_subcores=NS),
    compiler_params=pltpu.CompilerParams(needs_layout_passes=False,
                                         use_tc_tiling_on_sc=False),
    scratch_types=[
        pltpu.VMEM((SEQ_PER_W, L), jnp.int32),
        pltpu.VMEM((SEQ_PER_W * LCH,), jnp.int32),
        pltpu.VMEM((128, RP), jnp.float32),
        pltpu.VMEM((128, RP), jnp.float32),
        pltpu.VMEM((128, RP), jnp.float32),
        pltpu.VMEM((128, RP), jnp.float32),
        pltpu.VMEM((D, LCH, 129), jnp.float32),
        pltpu.VMEM((SEQ_PER_W,), jnp.int32),
        pltpu.SemaphoreType.DMA,
        pltpu.SemaphoreType.DMA,
        pltpu.SemaphoreType.DMA,
        pltpu.SemaphoreType.DMA,
        pltpu.SemaphoreType.DMA,
    ],
)
def _embedding_sc(x_hbm, table_hbm, emb_hbm, len_hbm, *rest):
    _sc_body(x_hbm, table_hbm, emb_hbm, len_hbm, *rest)


def kernel(x, table):
    tail = table[NTILECH * VCH:].reshape(-1)
    tbl_flat = _table_to_rowmajor(table.T, tail)
    emb_dlb, lengths = _embedding_sc(x, tbl_flat.reshape(V, RP))
    return jnp.transpose(emb_dlb, (2, 0, 1)), lengths


# speedup vs baseline: 1.3346x; 1.3346x over previous
import functools

import jax
import jax.numpy as jnp
from jax import lax
from jax.experimental import pallas as pl
from jax.experimental.pallas import tpu as pltpu
from jax.experimental.pallas import tpu_sc as plsc

B = 4096
L = 200
D = 32
V = 1000000
NC = 2
NS = 16
NW = NC * NS
SEQ_PER_W = B // NW
C0, C1 = 104, 96
UNROLL = 8

VCH = 512
RP = 40
NTILECH = V // VCH
VTAIL = V - NTILECH * VCH


def _fmt_body(tblT_hbm, tail_hbm, out_hbm, in0, in1, outb0, outb1,
              isem0, isem1, osem0, osem1):
    wid = lax.axis_index("s") * NC + lax.axis_index("c")

    lane = lax.iota(jnp.int32, 16)
    lane32 = lane * D

    in_bufs = (in0, in1)
    out_bufs = (outb0, outb1)
    isems = (isem0, isem1)
    osems = (osem0, osem1)

    def issue_read(c, inb, isem):
        v0 = c * VCH
        for dblk in range(4):
            for vblk in range(VCH // 128):
                pltpu.async_copy(
                    tblT_hbm.at[pl.ds(dblk * 8, 8),
                                pl.ds(v0 + vblk * 128, 128)],
                    inb.at[pl.ds(dblk * 8, 8), pl.ds(vblk * 128, 128)],
                    isem)

    def wait_read(inb, isem):
        for _ in range(4 * (VCH // 128)):
            pltpu.make_async_copy(
                tblT_hbm.at[pl.ds(0, 8), pl.ds(0, 128)],
                inb.at[pl.ds(0, 8), pl.ds(0, 128)], isem).wait()

    lane_rp = lane * RP

    def transpose_chunk(inb, outb):
        def tbody(g, carry):
            base = g * 16
            idx = lane_rp + base * RP
            for d in range(D):
                vals = inb[d, pl.ds(base, 16)]
                plsc.store_scatter(outb, [idx + d], vals)
            return carry
        lax.fori_loop(0, VCH // 16, tbody, 0)

    issue_read(wid, in0, isem0)
    issue_read(NW + wid, in1, isem1)

    def round_body(k, carry):
        for p in range(2):
            c = (2 * k + p) * NW + wid
            inb, outb = in_bufs[p], out_bufs[p]
            isem, osem = isems[p], osems[p]

            @pl.when(c < NTILECH)
            def _():
                wait_read(inb, isem)

                @pl.when(k > 0)
                def _():
                    pltpu.make_async_copy(
                        outb, out_hbm.at[pl.ds(0, VCH * RP)], osem).wait()

                transpose_chunk(inb, outb)
                pltpu.async_copy(outb, out_hbm.at[pl.ds(c * VCH * RP,
                                                        VCH * RP)], osem)

                @pl.when(c + 2 * NW < NTILECH)
                def _():
                    issue_read(c + 2 * NW, inb, isem)
        return carry

    nrounds = (NTILECH + 2 * NW - 1) // (2 * NW)
    lax.fori_loop(0, nrounds, round_body, 0)

    for p in range(2):
        pltpu.make_async_copy(out_bufs[p], out_hbm.at[pl.ds(0, VCH * RP)],
                              osems[p]).wait()

    @pl.when(wid == 0)
    def _():
        pltpu.sync_copy(tail_hbm, outb1.at[pl.ds(0, VTAIL * D)])

        def tailb(t, carry):
            for j in range(8):
                v = t * 8 + j
                outb0[pl.ds(v * RP, 16)] = outb1[pl.ds(v * D, 16)]
                outb0[pl.ds(v * RP + 16, 16)] = outb1[pl.ds(v * D + 16,
                                                            16)]
            return carry
        lax.fori_loop(0, VTAIL // 8, tailb, 0)
        pltpu.sync_copy(outb0.at[pl.ds(0, VTAIL * RP)],
                        out_hbm.at[pl.ds(NTILECH * VCH * RP, VTAIL * RP)])


@functools.partial(
    pl.kernel,
    out_type=jax.ShapeDtypeStruct((V * RP,), jnp.float32),
    mesh=plsc.VectorSubcoreMesh(core_axis_name="c", subcore_axis_name="s",
                                num_cores=NC, num_subcores=NS),
    compiler_params=pltpu.CompilerParams(needs_layout_passes=False,
                                         use_tc_tiling_on_sc=True),
    scratch_types=[
        pltpu.VMEM((D, VCH), jnp.float32),
        pltpu.VMEM((D, VCH), jnp.float32),
        pltpu.VMEM((VCH * RP,), jnp.float32),
        pltpu.VMEM((VCH * RP,), jnp.float32),
        pltpu.SemaphoreType.DMA,
        pltpu.SemaphoreType.DMA,
        pltpu.SemaphoreType.DMA,
        pltpu.SemaphoreType.DMA,
    ],
)
def _table_to_rowmajor(tblT_hbm, tail_hbm, out_hbm, *rest):
    _fmt_body(tblT_hbm, tail_hbm, out_hbm, *rest)


def _sc_body(x_hbm, table_hbm, emb_hbm, len_hbm,
             idx_v, rows0, rows1, outt0, outt1, len_v,
             gsem0, gsem1, osem0, osem1):
    wid = lax.axis_index("s") * NC + lax.axis_index("c")
    seq_base = wid * SEQ_PER_W

    pltpu.sync_copy(x_hbm.at[pl.ds(seq_base, SEQ_PER_W)], idx_v)

    lane = lax.iota(jnp.int32, 16)
    lane0 = lane == 0
    tail_mask = lane >= 8

    rows_bufs = (rows0, rows1)
    outt_bufs = (outt0, outt1)
    gsems = (gsem0, gsem1)
    osems = (osem0, osem1)

    def issue_gather(s_local, rows, gsem):
        pltpu.async_copy(table_hbm.at[idx_v.at[s_local, pl.ds(0, C0)]],
                         rows.at[pl.ds(0, C0)], gsem)
        pltpu.async_copy(table_hbm.at[idx_v.at[s_local, pl.ds(C0, C1)]],
                         rows.at[pl.ds(C0, C1)], gsem)

    def transpose_seq(rows, outt):
        def tbody(t, carry):
            for j in range(UNROLL):
                l = t * UNROLL + j
                v0 = rows[l, pl.ds(0, 16)]
                v1 = rows[l, pl.ds(16, 16)]
                lsplat = jnp.full((16,), l, jnp.int32)
                plsc.store_scatter(outt, [lane, lsplat], v0)
                plsc.store_scatter(outt, [lane + 16, lsplat], v1)
            return carry
        lax.fori_loop(0, L // UNROLL, tbody, 0)

    def count_seq(s_local):
        cnt = jnp.zeros((16,), jnp.int32)
        for j in range(12):
            v = idx_v[s_local, pl.ds(j * 16, 16)]
            cnt = cnt + (v != 0).astype(jnp.int32)
        v = idx_v[s_local, pl.ds(184, 16)]
        cnt = cnt + ((v != 0) & tail_mask).astype(jnp.int32)
        total = jnp.sum(cnt)
        plsc.store_scatter(len_v, [jnp.full((16,), s_local, jnp.int32)],
                           jnp.full((16,), total, jnp.int32), mask=lane0)

    issue_gather(0, rows0, gsem0)
    issue_gather(1, rows1, gsem1)

    def body(i, carry):
        for k in range(2):
            s = i * 2 + k
            rows, outt = rows_bufs[k], outt_bufs[k]
            gsem, osem = gsems[k], osems[k]

            pltpu.make_async_copy(table_hbm.at[pl.ds(0, L)], rows,
                                  gsem).wait()

            @pl.when(i > 0)
            def _():
                pltpu.make_async_copy(outt, emb_hbm.at[0], osem).wait()

            transpose_seq(rows, outt)
            count_seq(s)

            pltpu.async_copy(outt, emb_hbm.at[seq_base + s], osem)

            @pl.when(s + 2 < SEQ_PER_W)
            def _():
                issue_gather(s + 2, rows, gsem)
        return carry

    lax.fori_loop(0, SEQ_PER_W // 2, body, 0)

    for k in range(2):
        pltpu.make_async_copy(outt_bufs[k], emb_hbm.at[0], osems[k]).wait()
    pltpu.sync_copy(len_v, len_hbm.at[pl.ds(seq_base, SEQ_PER_W)])


@functools.partial(
    pl.kernel,
    out_type=(jax.ShapeDtypeStruct((B, D, L), jnp.float32),
              jax.ShapeDtypeStruct((B,), jnp.int32)),
    mesh=plsc.VectorSubcoreMesh(core_axis_name="c", subcore_axis_name="s",
                                num_cores=NC, num_subcores=NS),
    compiler_params=pltpu.CompilerParams(needs_layout_passes=False,
                                         use_tc_tiling_on_sc=False),
    scratch_types=[
        pltpu.VMEM((SEQ_PER_W, L), jnp.int32),
        pltpu.VMEM((L, RP), jnp.float32),
        pltpu.VMEM((L, RP), jnp.float32),
        pltpu.VMEM((D, L), jnp.float32),
        pltpu.VMEM((D, L), jnp.float32),
        pltpu.VMEM((SEQ_PER_W,), jnp.int32),
        pltpu.SemaphoreType.DMA,
        pltpu.SemaphoreType.DMA,
        pltpu.SemaphoreType.DMA,
        pltpu.SemaphoreType.DMA,
    ],
)
def _embedding_sc(x_hbm, table_hbm, emb_hbm, len_hbm, *rest):
    _sc_body(x_hbm, table_hbm, emb_hbm, len_hbm, *rest)


def kernel(x, table):
    tail = table[NTILECH * VCH:].reshape(-1)
    tbl_flat = _table_to_rowmajor(table.T, tail)
    return _embedding_sc(x, tbl_flat.reshape(V, RP))
